# baseline (speedup 1.0000x reference)
import functools

import jax
import jax.numpy as jnp
from jax import lax
from jax.experimental import pallas as pl
from jax.experimental.pallas import tpu as pltpu, tpu_sc as plsc

N = 32768
NG = 16
TAU = 0.2
GAMMA = 0.05
MAX_ITERS = 30
L16 = 16
SLICE = 2048
CH = 2048
BIG = 1e30

_mesh = plsc.VectorSubcoreMesh(
    core_axis_name="c", subcore_axis_name="s", num_cores=1)


@functools.partial(
    pl.kernel,
    out_type=(jax.ShapeDtypeStruct((N,), jnp.float32),
              jax.ShapeDtypeStruct((2 * L16, L16), jnp.float32),
              jax.ShapeDtypeStruct((L16, L16), jnp.float32)),
    mesh=_mesh,
    compiler_params=pltpu.CompilerParams(needs_layout_passes=False),
    scratch_types=[
        pltpu.VMEM((SLICE,), jnp.int32),
        pltpu.VMEM((SLICE,), jnp.float32),
        pltpu.VMEM((SLICE,), jnp.float32),
        pltpu.VMEM((SLICE,), jnp.float32),
        pltpu.VMEM((N,), jnp.float32),
        pltpu.VMEM((CH,), jnp.float32),
        pltpu.VMEM((CH,), jnp.float32),
        pltpu.VMEM((L16,), jnp.float32),
        pltpu.VMEM((L16,), jnp.float32),
        pltpu.VMEM((2, L16), jnp.float32),
        pltpu.VMEM((2 * L16, L16), jnp.float32),
        pltpu.VMEM((L16,), jnp.float32),
        pltpu.VMEM((L16, L16), jnp.float32),
        pltpu.VMEM((L16,), jnp.float32),
        pltpu.SemaphoreType.DMA,
        pltpu.SemaphoreType.DMA,
        pltpu.SemaphoreType.DMA,
    ],
)
def _projector(y_raw_hbm, y_real_hbm, gid_hbm, out_hbm, xch_a,
               xch_m, gbuf, rbuf, ebuf, obuf, abuf, craw, creal, cnt_v,
               s0_v, acc2_v, rows2_v, tmp_v, mrows_v, mtab_v,
               sem_g, sem_r, sem_e):
    s = lax.axis_index("s")
    iota = lax.iota(jnp.int32, L16)
    zeros_f = jnp.zeros((L16,), jnp.float32)
    zeros_i = jnp.zeros((L16,), jnp.int32)
    ones_f = jnp.ones((L16,), jnp.float32)
    ones_i = jnp.ones((L16,), jnp.int32)
    ob = s * SLICE

    cp_g = pltpu.async_copy(gid_hbm.at[pl.ds(ob, SLICE)], gbuf, sem_g)
    cp_r = pltpu.async_copy(y_raw_hbm.at[pl.ds(ob, SLICE)], rbuf, sem_r)
    cp_e = pltpu.async_copy(y_real_hbm.at[pl.ds(ob, SLICE)], ebuf, sem_e)
    cp_g.wait()
    cp_r.wait()
    cp_e.wait()

    cnt_v[...] = zeros_f
    s0_v[...] = zeros_f

    def phase_a(j, carry):
        for u in range(4):
            o = (4 * j + u) * L16
            gv = gbuf[pl.ds(o, L16)]
            vr = rbuf[pl.ds(o, L16)]
            vy = ebuf[pl.ds(o, L16)]
            yc = jnp.maximum(vy, 1e-9)
            a = vr / yc
            cv = jnp.minimum(jnp.maximum(a, 1.0 - TAU), 1.0 + TAU)
            plsc.addupdate_scatter(cnt_v, [gv], ones_f)
            plsc.addupdate_scatter(s0_v, [gv], cv)
        return carry

    lax.fori_loop(0, SLICE // (4 * L16), phase_a, 0)
    acc2_v[0] = cnt_v[...]
    acc2_v[1] = s0_v[...]
    pltpu.sync_copy(acc2_v, xch_a.at[pl.ds(2 * s, 2)])
    plsc.subcore_barrier()
    pltpu.sync_copy(xch_a, rows2_v)

    nf_v = rows2_v[0]
    s0t = rows2_v[1]
    for h in range(1, L16):
        nf_v = nf_v + rows2_v[2 * h]
        s0t = s0t + rows2_v[2 * h + 1]
    tot = nf_v.astype(jnp.int32)
    inb_v = (s0t >= (1.0 - GAMMA) * nf_v) & (s0t <= (1.0 + GAMMA) * nf_v)
    all_inb = jnp.all(inb_v)
    mtab_v[...] = zeros_f

    @pl.when(jnp.logical_not(all_inb))
    def _rare():
        start = jnp.sum(jnp.where(iota < s, tot, 0))
        n_g = jnp.sum(jnp.where(iota == s, tot, 0))
        end = start + n_g
        S0 = jnp.sum(jnp.where(iota == s, s0t, 0.0))
        nf = n_g.astype(jnp.float32)
        Lb = (1.0 - GAMMA) * nf
        Ub = (1.0 + GAMMA) * nf
        T = jnp.where(S0 < Lb, Lb, Ub)
        own_out = jnp.sum(
            jnp.where(iota == s,
                      jnp.logical_not(inb_v).astype(jnp.int32), 0)) > 0
        tmp_v[...] = zeros_f

        @pl.when(own_out)
        def _bisect():
            base = (start // CH) * CH
            stop = ((end + CH - 1) // CH) * CH
            nchunks = (stop - base) // CH

            def stage_chunk(k, carry):
                off = base + k * CH
                pltpu.sync_copy(y_raw_hbm.at[pl.ds(off, CH)], craw)
                pltpu.sync_copy(y_real_hbm.at[pl.ds(off, CH)], creal)

                def inner(j, car):
                    amina, amaxa = car
                    for u in range(8):
                        o = (8 * j + u) * L16
                        vr = craw[pl.ds(o, L16)]
                        vy = creal[pl.ds(o, L16)]
                        yc = jnp.maximum(vy, 1e-9)
                        a = vr / yc
                        idx0 = off + o
                        msk = (iota >= start - idx0) & (iota < end - idx0)
                        a_s = jnp.where(msk, a, BIG)
                        abuf[pl.ds(idx0, L16)] = a_s
                        amina = jnp.minimum(amina, a_s)
                        amaxa = jnp.maximum(amaxa, jnp.where(msk, a, -BIG))
                    return (amina, amaxa)

                return lax.fori_loop(0, CH // (8 * L16), inner, carry)

            amina, amaxa = lax.fori_loop(
                0, nchunks, stage_chunk,
                (jnp.full((L16,), BIG, jnp.float32),
                 jnp.full((L16,), -BIG, jnp.float32)))

            amin = jnp.min(amina)
            amax = jnp.max(amaxa)
            lo0 = ((1.0 - TAU) - amax) - 1.0
            hi0 = ((1.0 + TAU) - amin) + 1.0
            base_b = (start // (8 * L16)) * (8 * L16)
            stop_b = ((end + 8 * L16 - 1) // (8 * L16)) * (8 * L16)
            pad_b = (1.0 + TAU) * (stop_b - base_b - n_g).astype(jnp.float32)
            nv = (stop_b - base_b) // (8 * L16)

            def bis(it, carry):
                lo, hi, _ = carry
                mid = 0.5 * (lo + hi)

                def red(k, acc):
                    a0, a1, a2, a3 = acc
                    off = base_b + k * (8 * L16)
                    for u in range(8):
                        v = abuf[pl.ds(off + u * L16, L16)]
                        cv = jnp.minimum(
                            jnp.maximum(v + mid, 1.0 - TAU), 1.0 + TAU)
                        if u % 4 == 0:
                            a0 = a0 + cv
                        elif u % 4 == 1:
                            a1 = a1 + cv
                        elif u % 4 == 2:
                            a2 = a2 + cv
                        else:
                            a3 = a3 + cv
                    return (a0, a1, a2, a3)

                a0, a1, a2, a3 = lax.fori_loop(
                    0, nv, red, (zeros_f, zeros_f, zeros_f, zeros_f))
                Sm = jnp.sum((a0 + a1) + (a2 + a3)) - pad_b
                pred = Sm < T
                return (jnp.where(pred, mid, lo),
                        jnp.where(pred, hi, mid), mid)

            _, _, mid_last = lax.fori_loop(
                0, MAX_ITERS, bis, (lo0, hi0, jnp.float32(0.0)))
            tmp_v[...] = jnp.broadcast_to(mid_last, (L16,))

        pltpu.sync_copy(tmp_v, xch_m.at[s])
        plsc.subcore_barrier()
        pltpu.sync_copy(xch_m, mrows_v)
        mt = zeros_f
        for h in range(L16):
            mt = jnp.where(iota == h, mrows_v[h], mt)
        mtab_v[...] = mt

    def phase_out(j, carry):
        for u in range(4):
            o = (4 * j + u) * L16
            vr = rbuf[pl.ds(o, L16)]
            vy = ebuf[pl.ds(o, L16)]
            gv = gbuf[pl.ds(o, L16)]
            yc = jnp.maximum(vy, 1e-9)
            a = vr / yc
            mv = plsc.load_gather(mtab_v, [gv])
            res = yc * jnp.minimum(jnp.maximum(a + mv, 1.0 - TAU), 1.0 + TAU)
            obuf[pl.ds(o, L16)] = res
        return carry

    lax.fori_loop(0, SLICE // (4 * L16), phase_out, 0)
    pltpu.sync_copy(obuf, out_hbm.at[pl.ds(ob, SLICE)])


def kernel(y_raw, y_real, group_ids, n_groups):
    del n_groups
    if group_ids.dtype != jnp.int32:
        group_ids = group_ids.astype(jnp.int32)
    out, _, _ = _projector(y_raw, y_real, group_ids)
    return out


# speedup vs baseline: 1.0304x; 1.0304x over previous
import functools

import jax
import jax.numpy as jnp
from jax import lax
from jax.experimental import pallas as pl
from jax.experimental.pallas import tpu as pltpu, tpu_sc as plsc

N = 32768
NG = 16
TAU = 0.2
GAMMA = 0.05
MAX_ITERS = 30
L16 = 16
SLICE = 2048
CH = 2048
BIG = 1e30

_mesh = plsc.VectorSubcoreMesh(
    core_axis_name="c", subcore_axis_name="s", num_cores=1)


@functools.partial(
    pl.kernel,
    out_type=(jax.ShapeDtypeStruct((N,), jnp.float32),
              jax.ShapeDtypeStruct((2 * L16, L16), jnp.float32),
              jax.ShapeDtypeStruct((L16, L16), jnp.float32)),
    mesh=_mesh,
    compiler_params=pltpu.CompilerParams(needs_layout_passes=False),
    scratch_types=[
        pltpu.VMEM((SLICE,), jnp.int32),
        pltpu.VMEM((SLICE,), jnp.float32),
        pltpu.VMEM((SLICE,), jnp.float32),
        pltpu.VMEM((SLICE,), jnp.float32),
        pltpu.VMEM((N,), jnp.float32),
        pltpu.VMEM((CH,), jnp.float32),
        pltpu.VMEM((CH,), jnp.float32),
        pltpu.VMEM((L16,), jnp.float32),
        pltpu.VMEM((L16,), jnp.float32),
        pltpu.VMEM((2, L16), jnp.float32),
        pltpu.VMEM((2 * L16, L16), jnp.float32),
        pltpu.VMEM((L16,), jnp.float32),
        pltpu.VMEM((L16, L16), jnp.float32),
        pltpu.VMEM((L16,), jnp.float32),
        pltpu.SemaphoreType.DMA,
        pltpu.SemaphoreType.DMA,
        pltpu.SemaphoreType.DMA,
    ],
)
def _projector(y_raw_hbm, y_real_hbm, gid_hbm, out_hbm, xch_a,
               xch_m, gbuf, rbuf, ebuf, obuf, abuf, craw, creal, cnt_v,
               s0_v, acc2_v, rows2_v, tmp_v, mrows_v, mtab_v,
               sem_g, sem_r, sem_e):
    s = lax.axis_index("s")
    iota = lax.iota(jnp.int32, L16)
    zeros_f = jnp.zeros((L16,), jnp.float32)
    zeros_i = jnp.zeros((L16,), jnp.int32)
    ones_f = jnp.ones((L16,), jnp.float32)
    ones_i = jnp.ones((L16,), jnp.int32)
    ob = s * SLICE

    cp_g = pltpu.async_copy(gid_hbm.at[pl.ds(ob, SLICE)], gbuf, sem_g)
    cp_r = pltpu.async_copy(y_raw_hbm.at[pl.ds(ob, SLICE)], rbuf, sem_r)
    cp_e = pltpu.async_copy(y_real_hbm.at[pl.ds(ob, SLICE)], ebuf, sem_e)
    cp_g.wait()
    cp_r.wait()
    cp_e.wait()

    cnt_v[...] = zeros_f
    s0_v[...] = zeros_f

    def phase_a(j, carry):
        for u in range(4):
            o = (4 * j + u) * L16
            gv = gbuf[pl.ds(o, L16)]
            vr = rbuf[pl.ds(o, L16)]
            vy = ebuf[pl.ds(o, L16)]
            yc = jnp.maximum(vy, 1e-9)
            a = vr / yc
            cv = jnp.minimum(jnp.maximum(a, 1.0 - TAU), 1.0 + TAU)
            plsc.addupdate_scatter(cnt_v, [gv], ones_f)
            plsc.addupdate_scatter(s0_v, [gv], cv)
            obuf[pl.ds(o, L16)] = yc * cv
        return carry

    lax.fori_loop(0, SLICE // (4 * L16), phase_a, 0)
    acc2_v[0] = cnt_v[...]
    acc2_v[1] = s0_v[...]
    pltpu.sync_copy(acc2_v, xch_a.at[pl.ds(2 * s, 2)])
    plsc.subcore_barrier()
    pltpu.sync_copy(xch_a, rows2_v)

    nf_v = rows2_v[0]
    s0t = rows2_v[1]
    for h in range(1, L16):
        nf_v = nf_v + rows2_v[2 * h]
        s0t = s0t + rows2_v[2 * h + 1]
    tot = nf_v.astype(jnp.int32)
    inb_v = (s0t >= (1.0 - GAMMA) * nf_v) & (s0t <= (1.0 + GAMMA) * nf_v)
    all_inb = jnp.all(inb_v)

    @pl.when(jnp.logical_not(all_inb))
    def _rare():
        start = jnp.sum(jnp.where(iota < s, tot, 0))
        n_g = jnp.sum(jnp.where(iota == s, tot, 0))
        end = start + n_g
        S0 = jnp.sum(jnp.where(iota == s, s0t, 0.0))
        nf = n_g.astype(jnp.float32)
        Lb = (1.0 - GAMMA) * nf
        Ub = (1.0 + GAMMA) * nf
        T = jnp.where(S0 < Lb, Lb, Ub)
        own_out = jnp.sum(
            jnp.where(iota == s,
                      jnp.logical_not(inb_v).astype(jnp.int32), 0)) > 0
        tmp_v[...] = zeros_f

        @pl.when(own_out)
        def _bisect():
            base = (start // CH) * CH
            stop = ((end + CH - 1) // CH) * CH
            nchunks = (stop - base) // CH

            def stage_chunk(k, carry):
                off = base + k * CH
                pltpu.sync_copy(y_raw_hbm.at[pl.ds(off, CH)], craw)
                pltpu.sync_copy(y_real_hbm.at[pl.ds(off, CH)], creal)

                def inner(j, car):
                    amina, amaxa = car
                    for u in range(8):
                        o = (8 * j + u) * L16
                        vr = craw[pl.ds(o, L16)]
                        vy = creal[pl.ds(o, L16)]
                        yc = jnp.maximum(vy, 1e-9)
                        a = vr / yc
                        idx0 = off + o
                        msk = (iota >= start - idx0) & (iota < end - idx0)
                        a_s = jnp.where(msk, a, BIG)
                        abuf[pl.ds(idx0, L16)] = a_s
                        amina = jnp.minimum(amina, a_s)
                        amaxa = jnp.maximum(amaxa, jnp.where(msk, a, -BIG))
                    return (amina, amaxa)

                return lax.fori_loop(0, CH // (8 * L16), inner, carry)

            amina, amaxa = lax.fori_loop(
                0, nchunks, stage_chunk,
                (jnp.full((L16,), BIG, jnp.float32),
                 jnp.full((L16,), -BIG, jnp.float32)))

            amin = jnp.min(amina)
            amax = jnp.max(amaxa)
            lo0 = ((1.0 - TAU) - amax) - 1.0
            hi0 = ((1.0 + TAU) - amin) + 1.0
            base_b = (start // (8 * L16)) * (8 * L16)
            stop_b = ((end + 8 * L16 - 1) // (8 * L16)) * (8 * L16)
            pad_b = (1.0 + TAU) * (stop_b - base_b - n_g).astype(jnp.float32)
            nv = (stop_b - base_b) // (8 * L16)

            def bis(it, carry):
                lo, hi, _ = carry
                mid = 0.5 * (lo + hi)

                def red(k, acc):
                    a0, a1, a2, a3 = acc
                    off = base_b + k * (8 * L16)
                    for u in range(8):
                        v = abuf[pl.ds(off + u * L16, L16)]
                        cv = jnp.minimum(
                            jnp.maximum(v + mid, 1.0 - TAU), 1.0 + TAU)
                        if u % 4 == 0:
                            a0 = a0 + cv
                        elif u % 4 == 1:
                            a1 = a1 + cv
                        elif u % 4 == 2:
                            a2 = a2 + cv
                        else:
                            a3 = a3 + cv
                    return (a0, a1, a2, a3)

                a0, a1, a2, a3 = lax.fori_loop(
                    0, nv, red, (zeros_f, zeros_f, zeros_f, zeros_f))
                Sm = jnp.sum((a0 + a1) + (a2 + a3)) - pad_b
                pred = Sm < T
                return (jnp.where(pred, mid, lo),
                        jnp.where(pred, hi, mid), mid)

            _, _, mid_last = lax.fori_loop(
                0, MAX_ITERS, bis, (lo0, hi0, jnp.float32(0.0)))
            tmp_v[...] = jnp.broadcast_to(mid_last, (L16,))

        pltpu.sync_copy(tmp_v, xch_m.at[s])
        plsc.subcore_barrier()
        pltpu.sync_copy(xch_m, mrows_v)
        mt = zeros_f
        for h in range(L16):
            mt = jnp.where(iota == h, mrows_v[h], mt)
        mtab_v[...] = mt

        def phase_out(j, carry):
            for u in range(4):
                o = (4 * j + u) * L16
                vr = rbuf[pl.ds(o, L16)]
                vy = ebuf[pl.ds(o, L16)]
                gv = gbuf[pl.ds(o, L16)]
                yc = jnp.maximum(vy, 1e-9)
                a = vr / yc
                mv = plsc.load_gather(mtab_v, [gv])
                res = yc * jnp.minimum(
                    jnp.maximum(a + mv, 1.0 - TAU), 1.0 + TAU)
                obuf[pl.ds(o, L16)] = res
            return carry

        lax.fori_loop(0, SLICE // (4 * L16), phase_out, 0)

    pltpu.sync_copy(obuf, out_hbm.at[pl.ds(ob, SLICE)])


def kernel(y_raw, y_real, group_ids, n_groups):
    del n_groups
    if group_ids.dtype != jnp.int32:
        group_ids = group_ids.astype(jnp.int32)
    out, _, _ = _projector(y_raw, y_real, group_ids)
    return out


# speedup vs baseline: 1.0380x; 1.0074x over previous
import functools

import jax
import jax.numpy as jnp
from jax import lax
from jax.experimental import pallas as pl
from jax.experimental.pallas import tpu as pltpu, tpu_sc as plsc

N = 32768
NG = 16
TAU = 0.2
GAMMA = 0.05
MAX_ITERS = 30
L16 = 16
SLICE = 2048
CH = 2048
BIG = 1e30

_mesh = plsc.VectorSubcoreMesh(
    core_axis_name="c", subcore_axis_name="s", num_cores=1)


@functools.partial(
    pl.kernel,
    out_type=(jax.ShapeDtypeStruct((N,), jnp.float32),
              jax.ShapeDtypeStruct((2 * L16, L16), jnp.float32),
              jax.ShapeDtypeStruct((L16, L16), jnp.float32)),
    mesh=_mesh,
    compiler_params=pltpu.CompilerParams(needs_layout_passes=False),
    scratch_types=[
        pltpu.VMEM((SLICE,), jnp.int32),
        pltpu.VMEM((SLICE,), jnp.float32),
        pltpu.VMEM((SLICE,), jnp.float32),
        pltpu.VMEM((SLICE,), jnp.float32),
        pltpu.VMEM((N,), jnp.float32),
        pltpu.VMEM((CH,), jnp.float32),
        pltpu.VMEM((CH,), jnp.float32),
        pltpu.VMEM((L16,), jnp.float32),
        pltpu.VMEM((L16,), jnp.float32),
        pltpu.VMEM((2, L16), jnp.float32),
        pltpu.VMEM((2 * L16, L16), jnp.float32),
        pltpu.VMEM((L16,), jnp.float32),
        pltpu.VMEM((L16, L16), jnp.float32),
        pltpu.VMEM((L16,), jnp.float32),
        pltpu.SemaphoreType.DMA,
        pltpu.SemaphoreType.DMA,
        pltpu.SemaphoreType.DMA,
        pltpu.SemaphoreType.DMA,
    ],
)
def _projector(y_raw_hbm, y_real_hbm, gid_hbm, out_hbm, xch_a,
               xch_m, gbuf, rbuf, ebuf, obuf, abuf, craw, creal, cnt_v,
               s0_v, acc2_v, rows2_v, tmp_v, mrows_v, mtab_v,
               sem_g, sem_r, sem_e, sem_o):
    s = lax.axis_index("s")
    iota = lax.iota(jnp.int32, L16)
    zeros_f = jnp.zeros((L16,), jnp.float32)
    zeros_i = jnp.zeros((L16,), jnp.int32)
    ones_f = jnp.ones((L16,), jnp.float32)
    ones_i = jnp.ones((L16,), jnp.int32)
    ob = s * SLICE

    cp_g = pltpu.async_copy(gid_hbm.at[pl.ds(ob, SLICE)], gbuf, sem_g)
    cp_r = pltpu.async_copy(y_raw_hbm.at[pl.ds(ob, SLICE)], rbuf, sem_r)
    cp_e = pltpu.async_copy(y_real_hbm.at[pl.ds(ob, SLICE)], ebuf, sem_e)
    cp_g.wait()
    cp_r.wait()
    cp_e.wait()

    cnt_v[...] = zeros_f
    s0_v[...] = zeros_f

    def phase_a(j, carry):
        for u in range(4):
            o = (4 * j + u) * L16
            gv = gbuf[pl.ds(o, L16)]
            vr = rbuf[pl.ds(o, L16)]
            vy = ebuf[pl.ds(o, L16)]
            yc = jnp.maximum(vy, 1e-9)
            a = vr / yc
            cv = jnp.minimum(jnp.maximum(a, 1.0 - TAU), 1.0 + TAU)
            plsc.addupdate_scatter(cnt_v, [gv], ones_f)
            plsc.addupdate_scatter(s0_v, [gv], cv)
            obuf[pl.ds(o, L16)] = yc * cv
        return carry

    lax.fori_loop(0, SLICE // (4 * L16), phase_a, 0)
    cp_o = pltpu.async_copy(obuf, out_hbm.at[pl.ds(ob, SLICE)], sem_o)
    acc2_v[0] = cnt_v[...]
    acc2_v[1] = s0_v[...]
    pltpu.sync_copy(acc2_v, xch_a.at[pl.ds(2 * s, 2)])
    plsc.subcore_barrier()
    pltpu.sync_copy(xch_a, rows2_v)
    cp_o.wait()

    nf_v = rows2_v[0]
    s0t = rows2_v[1]
    for h in range(1, L16):
        nf_v = nf_v + rows2_v[2 * h]
        s0t = s0t + rows2_v[2 * h + 1]
    tot = nf_v.astype(jnp.int32)
    inb_v = (s0t >= (1.0 - GAMMA) * nf_v) & (s0t <= (1.0 + GAMMA) * nf_v)
    all_inb = jnp.all(inb_v)

    @pl.when(jnp.logical_not(all_inb))
    def _rare():
        start = jnp.sum(jnp.where(iota < s, tot, 0))
        n_g = jnp.sum(jnp.where(iota == s, tot, 0))
        end = start + n_g
        S0 = jnp.sum(jnp.where(iota == s, s0t, 0.0))
        nf = n_g.astype(jnp.float32)
        Lb = (1.0 - GAMMA) * nf
        Ub = (1.0 + GAMMA) * nf
        T = jnp.where(S0 < Lb, Lb, Ub)
        own_out = jnp.sum(
            jnp.where(iota == s,
                      jnp.logical_not(inb_v).astype(jnp.int32), 0)) > 0
        tmp_v[...] = zeros_f

        @pl.when(own_out)
        def _bisect():
            base = (start // CH) * CH
            stop = ((end + CH - 1) // CH) * CH
            nchunks = (stop - base) // CH

            def stage_chunk(k, carry):
                off = base + k * CH
                pltpu.sync_copy(y_raw_hbm.at[pl.ds(off, CH)], craw)
                pltpu.sync_copy(y_real_hbm.at[pl.ds(off, CH)], creal)

                def inner(j, car):
                    amina, amaxa = car
                    for u in range(8):
                        o = (8 * j + u) * L16
                        vr = craw[pl.ds(o, L16)]
                        vy = creal[pl.ds(o, L16)]
                        yc = jnp.maximum(vy, 1e-9)
                        a = vr / yc
                        idx0 = off + o
                        msk = (iota >= start - idx0) & (iota < end - idx0)
                        a_s = jnp.where(msk, a, BIG)
                        abuf[pl.ds(idx0, L16)] = a_s
                        amina = jnp.minimum(amina, a_s)
                        amaxa = jnp.maximum(amaxa, jnp.where(msk, a, -BIG))
                    return (amina, amaxa)

                return lax.fori_loop(0, CH // (8 * L16), inner, carry)

            amina, amaxa = lax.fori_loop(
                0, nchunks, stage_chunk,
                (jnp.full((L16,), BIG, jnp.float32),
                 jnp.full((L16,), -BIG, jnp.float32)))

            amin = jnp.min(amina)
            amax = jnp.max(amaxa)
            lo0 = ((1.0 - TAU) - amax) - 1.0
            hi0 = ((1.0 + TAU) - amin) + 1.0
            base_b = (start // (8 * L16)) * (8 * L16)
            stop_b = ((end + 8 * L16 - 1) // (8 * L16)) * (8 * L16)
            pad_b = (1.0 + TAU) * (stop_b - base_b - n_g).astype(jnp.float32)
            nv = (stop_b - base_b) // (8 * L16)

            def bis(it, carry):
                lo, hi, _ = carry
                mid = 0.5 * (lo + hi)

                def red(k, acc):
                    a0, a1, a2, a3 = acc
                    off = base_b + k * (8 * L16)
                    for u in range(8):
                        v = abuf[pl.ds(off + u * L16, L16)]
                        cv = jnp.minimum(
                            jnp.maximum(v + mid, 1.0 - TAU), 1.0 + TAU)
                        if u % 4 == 0:
                            a0 = a0 + cv
                        elif u % 4 == 1:
                            a1 = a1 + cv
                        elif u % 4 == 2:
                            a2 = a2 + cv
                        else:
                            a3 = a3 + cv
                    return (a0, a1, a2, a3)

                a0, a1, a2, a3 = lax.fori_loop(
                    0, nv, red, (zeros_f, zeros_f, zeros_f, zeros_f))
                Sm = jnp.sum((a0 + a1) + (a2 + a3)) - pad_b
                pred = Sm < T
                return (jnp.where(pred, mid, lo),
                        jnp.where(pred, hi, mid), mid)

            _, _, mid_last = lax.fori_loop(
                0, MAX_ITERS, bis, (lo0, hi0, jnp.float32(0.0)))
            tmp_v[...] = jnp.broadcast_to(mid_last, (L16,))

        pltpu.sync_copy(tmp_v, xch_m.at[s])
        plsc.subcore_barrier()
        pltpu.sync_copy(xch_m, mrows_v)
        mt = zeros_f
        for h in range(L16):
            mt = jnp.where(iota == h, mrows_v[h], mt)
        mtab_v[...] = mt

        def phase_out(j, carry):
            for u in range(4):
                o = (4 * j + u) * L16
                vr = rbuf[pl.ds(o, L16)]
                vy = ebuf[pl.ds(o, L16)]
                gv = gbuf[pl.ds(o, L16)]
                yc = jnp.maximum(vy, 1e-9)
                a = vr / yc
                mv = plsc.load_gather(mtab_v, [gv])
                res = yc * jnp.minimum(
                    jnp.maximum(a + mv, 1.0 - TAU), 1.0 + TAU)
                obuf[pl.ds(o, L16)] = res
            return carry

        lax.fori_loop(0, SLICE // (4 * L16), phase_out, 0)
        pltpu.sync_copy(obuf, out_hbm.at[pl.ds(ob, SLICE)])


def kernel(y_raw, y_real, group_ids, n_groups):
    del n_groups
    if group_ids.dtype != jnp.int32:
        group_ids = group_ids.astype(jnp.int32)
    out, _, _ = _projector(y_raw, y_real, group_ids)
    return out
